# baseline (speedup 1.0000x reference)
import functools

import jax
import jax.numpy as jnp
from jax import lax
from jax.experimental import pallas as pl
from jax.experimental.pallas import tpu as pltpu
from jax.experimental.pallas import tpu_sc as plsc

NUM_SC_CORES = 2
NUM_SUBCORES = 16
NUM_WORKERS = NUM_SC_CORES * NUM_SUBCORES
ROW_PAD = 16
ROW_OUT = 12
CHUNK = 128


def _expmap_table_body(pose_ref, out_ref):
    tv = pose_ref[...].T
    tx, ty, tz = tv[0:1], tv[1:2], tv[2:3]
    ax, ay, az = tv[3:4], tv[4:5], tv[5:6]
    theta2 = ax * ax + ay * ay + az * az
    theta = jnp.sqrt(theta2)
    near = theta < 1e-2
    theta_nz = jnp.where(near, 1.0, theta)
    theta2_nz = jnp.where(near, 1.0, theta2)
    sine = jnp.sin(theta)
    cosine = jnp.where(near, 8.0 / (4.0 + theta2) - 1.0, jnp.cos(theta))
    sbt = jnp.where(near, 0.5 * cosine + 0.5, sine / theta_nz)
    omc = jnp.where(near, 0.5 * sbt, (1.0 - cosine) / theta2_nz)
    wx, wy, wz = sbt * ax, sbt * ay, sbt * az
    r00 = omc * ax * ax + cosine
    r01 = omc * ax * ay - wz
    r02 = omc * ax * az + wy
    r10 = omc * ay * ax + wz
    r11 = omc * ay * ay + cosine
    r12 = omc * ay * az - wx
    r20 = omc * az * ax - wy
    r21 = omc * az * ay + wx
    r22 = omc * az * az + cosine
    zero = jnp.zeros_like(r00)
    table_t = jnp.concatenate(
        [r00, r01, r02, tx, r10, r11, r12, ty, r20, r21, r22, tz,
         zero, zero, zero, zero], axis=0)
    out_ref[...] = table_t.T


def _expmap_table(pose):
    n = pose.shape[0]
    return pl.pallas_call(
        _expmap_table_body,
        out_shape=jax.ShapeDtypeStruct((n, ROW_PAD), jnp.float32),
    )(pose)


def _make_sc_gather(batch):
    b_per_w = batch // NUM_WORKERS
    n_chunks = b_per_w // CHUNK
    mesh = plsc.VectorSubcoreMesh(core_axis_name="c", subcore_axis_name="s")

    f32_per_w = b_per_w * ROW_OUT
    n_vregs = b_per_w * ROW_OUT // 16
    group = 24

    @functools.partial(
        pl.kernel,
        out_type=jax.ShapeDtypeStruct((batch * ROW_OUT,), jnp.float32),
        mesh=mesh,
        compiler_params=pltpu.CompilerParams(
            use_tc_tiling_on_sc=False, needs_layout_passes=False),
        scratch_types=[
            pltpu.VMEM((b_per_w,), jnp.int32),
            pltpu.VMEM((b_per_w, ROW_PAD), jnp.float32),
            pltpu.VMEM((f32_per_w,), jnp.float32),
            pltpu.SemaphoreType.DMA,
        ],
    )
    def gather(table_hbm, idx_hbm, out_hbm, idx_v, rows_v, out_v, sem):
        wid = lax.axis_index("s") * NUM_SC_CORES + lax.axis_index("c")
        pltpu.sync_copy(idx_hbm.at[pl.ds(wid * b_per_w, b_per_w)], idx_v)
        copies = [
            pltpu.async_copy(
                table_hbm.at[idx_v.at[pl.ds(j * CHUNK, CHUNK)]],
                rows_v.at[pl.ds(j * CHUNK, CHUNK)],
                sem,
            )
            for j in range(n_chunks)
        ]
        for c in copies:
            c.wait()
        lane = lax.iota(jnp.int32, 16)
        e_base, c_base = [], []
        for i in range(group):
            off = (16 * i) % ROW_OUT
            q0 = (16 * i) // ROW_OUT
            x = lane + off
            carry = (jnp.minimum(jnp.maximum(x - (ROW_OUT - 1), 0), 1)
                     + jnp.minimum(jnp.maximum(x - (2 * ROW_OUT - 1), 0), 1))
            e_base.append(q0 + carry)
            c_base.append(x - ROW_OUT * carry)

        def body(w):
            s = (group * 16 // ROW_OUT) * w
            base = group * 16 * w
            for i in range(group):
                res = plsc.load_gather(rows_v, [e_base[i] + s, c_base[i]])
                out_v[pl.ds(base + 16 * i, 16)] = res

        pl.loop(0, n_vregs // group)(body)
        pltpu.sync_copy(out_v, out_hbm.at[pl.ds(wid * f32_per_w, f32_per_w)])

    return gather


def kernel(indices, pose_adjustment):
    batch = indices.shape[0]
    table = _expmap_table(pose_adjustment.astype(jnp.float32))
    idx = indices.astype(jnp.int32)
    out = _make_sc_gather(batch)(table, idx)
    return out.reshape(batch, 3, 4)


# speedup vs baseline: 1.9877x; 1.9877x over previous
import functools

import jax
import jax.numpy as jnp
from jax import lax
from jax.experimental import pallas as pl
from jax.experimental.pallas import tpu as pltpu
from jax.experimental.pallas import tpu_sc as plsc

NUM_SC_CORES = 2
NUM_SUBCORES = 16
NUM_WORKERS = NUM_SC_CORES * NUM_SUBCORES
ROW_PAD = 16
ROW_OUT = 12
CHUNK = 128


def _expmap_table_body(pose_ref, out_ref):
    tv = pose_ref[...].T
    tx, ty, tz = tv[0:1], tv[1:2], tv[2:3]
    ax, ay, az = tv[3:4], tv[4:5], tv[5:6]
    theta2 = ax * ax + ay * ay + az * az
    theta = jnp.sqrt(theta2)
    near = theta < 1e-2
    theta_nz = jnp.where(near, 1.0, theta)
    theta2_nz = jnp.where(near, 1.0, theta2)
    sine = jnp.sin(theta)
    cosine = jnp.where(near, 8.0 / (4.0 + theta2) - 1.0, jnp.cos(theta))
    sbt = jnp.where(near, 0.5 * cosine + 0.5, sine / theta_nz)
    omc = jnp.where(near, 0.5 * sbt, (1.0 - cosine) / theta2_nz)
    wx, wy, wz = sbt * ax, sbt * ay, sbt * az
    r00 = omc * ax * ax + cosine
    r01 = omc * ax * ay - wz
    r02 = omc * ax * az + wy
    r10 = omc * ay * ax + wz
    r11 = omc * ay * ay + cosine
    r12 = omc * ay * az - wx
    r20 = omc * az * ax - wy
    r21 = omc * az * ay + wx
    r22 = omc * az * az + cosine
    zero = jnp.zeros_like(r00)
    table_t = jnp.concatenate(
        [r00, r01, r02, tx, r10, r11, r12, ty, r20, r21, r22, tz,
         zero, zero, zero, zero], axis=0)
    out_ref[...] = table_t.T


def _expmap_table(pose):
    n = pose.shape[0]
    return pl.pallas_call(
        _expmap_table_body,
        out_shape=jax.ShapeDtypeStruct((n, ROW_PAD), jnp.float32),
    )(pose)


def _make_sc_gather(batch):
    b_per_w = batch // NUM_WORKERS
    n_chunks = b_per_w // CHUNK
    mesh = plsc.VectorSubcoreMesh(core_axis_name="c", subcore_axis_name="s")

    unroll = 16

    @functools.partial(
        pl.kernel,
        out_type=jax.ShapeDtypeStruct((batch, ROW_OUT), jnp.float32),
        mesh=mesh,
        compiler_params=pltpu.CompilerParams(
            use_tc_tiling_on_sc=False, needs_layout_passes=False),
        scratch_types=[
            pltpu.VMEM((n_chunks, CHUNK), jnp.int32),
            pltpu.VMEM((b_per_w, ROW_PAD), jnp.float32),
            pltpu.VMEM((b_per_w, ROW_OUT), jnp.float32),
            pltpu.SemaphoreType.DMA,
        ],
    )
    def gather(table_hbm, idx_hbm, out_hbm, idx_v, rows_v, out_v, sem):
        wid = lax.axis_index("s") * NUM_SC_CORES + lax.axis_index("c")
        pltpu.sync_copy(idx_hbm.at[pl.ds(wid * n_chunks, n_chunks)], idx_v)
        copies = [
            pltpu.async_copy(
                table_hbm.at[idx_v.at[j]],
                rows_v.at[pl.ds(j * CHUNK, CHUNK)],
                sem,
            )
            for j in range(n_chunks)
        ]
        for c in copies:
            c.wait()
        lane = lax.iota(jnp.int32, 16)
        keep = lane < ROW_OUT

        def body(w):
            for i in range(unroll):
                u = unroll * w + i
                row = jnp.broadcast_to(u, (16,)).astype(jnp.int32)
                plsc.store_scatter(out_v, [row, lane], rows_v[u], mask=keep)

        pl.loop(0, b_per_w // unroll)(body)
        pltpu.sync_copy(out_v, out_hbm.at[pl.ds(wid * b_per_w, b_per_w)])

    return gather


def kernel(indices, pose_adjustment):
    batch = indices.shape[0]
    table = _expmap_table(pose_adjustment.astype(jnp.float32))
    idx = indices.astype(jnp.int32).reshape(batch // 128, 128)
    out = _make_sc_gather(batch)(table, idx)
    return out.reshape(batch, 3, 4)
